# Initial kernel scaffold; baseline (speedup 1.0000x reference)
import jax
import jax.numpy as jnp
from jax.experimental import pallas as pl


def kernel(x, pos, W_pe, b_pe, W_qkv, b_qkv, W_proj, b_proj, num_batches):
    raise NotImplementedError("write your pallas kernel here")


# speedup vs baseline: 36520.7075x; 36520.7075x over previous
import jax
import jax.numpy as jnp
import numpy as np
from jax.experimental import pallas as pl
from jax.experimental.pallas import tpu as pltpu

_DIM = 128
_H = 8
_E = _DIM // _H
_M = 16
_TOPK = 2
_T = 4096
_B = 2
_N = _T // (_B * _M)
_NB = _T // _M
_TB = _T // _B
_SCALE = 1.0 / (_E ** 0.5)

_F32 = jnp.float32


def _nt(a, b):
    return jax.lax.dot_general(
        a, b, (((1,), (1,)), ((), ())), preferred_element_type=_F32)


def _nn(a, b):
    return jax.lax.dot_general(
        a, b, (((1,), (0,)), ((), ())), preferred_element_type=_F32)


def _kernel(x_ref, posp_ref, wpe_ref, bpe_ref, wq_ref, bq_ref,
            wk0_ref, wk1_ref, wv0_ref, wv1_ref,
            bk0_ref, bk1_ref, bv0_ref, bv1_ref,
            wproj_ref, bproj_ref, out_ref):
    posp = posp_ref[:]
    pp = _nn(posp, wpe_ref[:])

    r = jax.lax.broadcasted_iota(jnp.int32, (_NB, _T), 0)
    c = jax.lax.broadcasted_iota(jnp.int32, (_NB, _T), 1)
    avg_t = jnp.where(r == c // _M, 1.0 / _M, 0.0).astype(_F32)
    r2 = jax.lax.broadcasted_iota(jnp.int32, (_T, _NB), 0)
    c2 = jax.lax.broadcasted_iota(jnp.int32, (_T, _NB), 1)
    member = jnp.where(r2 // _M == c2, 1.0, 0.0).astype(_F32)
    means = _nn(avg_t, pp)
    xp = x_ref[:] + pp - _nn(member, means) + bpe_ref[:]

    q = _nn(xp, wq_ref[:]) + bq_ref[:]

    wproj = wproj_ref[:]
    for bb in range(_B):
        base = bb * _TB
        xkv = xp[base:base + _H * _M, :]
        kmat0 = _nn(xkv, wk0_ref[:]) + bk0_ref[:]
        kmat1 = _nn(xkv, wk1_ref[:]) + bk1_ref[:]
        vmat0 = _nn(xkv, wv0_ref[:]) + bv0_ref[:]
        vmat1 = _nn(xkv, wv1_ref[:]) + bv1_ref[:]
        qb = q[base:base + _TB, :]
        acc = jnp.broadcast_to(bproj_ref[:], (_TB, _DIM))
        for h in range(_H):
            qh = qb[:, h * _E:(h + 1) * _E]
            k0 = kmat0[h * _M:(h + 1) * _M, :]
            k1 = kmat1[h * _M:(h + 1) * _M, :]
            v0 = vmat0[h * _M:(h + 1) * _M, :]
            v1 = vmat1[h * _M:(h + 1) * _M, :]
            s0 = _nt(qh, k0) * _SCALE
            s1 = _nt(qh, k1) * _SCALE
            mx = jnp.maximum(jnp.max(s0, axis=1, keepdims=True),
                             jnp.max(s1, axis=1, keepdims=True))
            e0 = jnp.exp(s0 - mx)
            e1 = jnp.exp(s1 - mx)
            den = (jnp.sum(e0, axis=1, keepdims=True) +
                   jnp.sum(e1, axis=1, keepdims=True))
            o = (_nn(e0, v0) + _nn(e1, v1)) / den
            acc = acc + _nn(o, wproj[h * _E:(h + 1) * _E, :])
        out_ref[base:base + _TB, :] = acc


def kernel(x, pos, W_pe, b_pe, W_qkv, b_qkv, W_proj, b_proj, num_batches):
    del num_batches
    x = x.astype(_F32)
    dimy = pos.shape[-1]

    posp = jnp.pad(pos.astype(_F32), ((0, 0), (0, _DIM - dimy)))
    wpe = jnp.pad(W_pe.astype(_F32).T, ((0, _DIM - dimy), (0, 0)))

    hh = np.arange(_H)[:, None]
    ee = np.arange(_E)[None, :]
    q_rows = (hh * (_E * 3) + ee * 3 + 0).reshape(-1)
    wq = W_qkv[q_rows].astype(_F32).T
    bq = b_qkv[q_rows].astype(_F32)[None, :]

    def kv_w(head, s):
        rows = (head * (_E * 3) + np.arange(_E) * 3 + s)
        return W_qkv[rows].astype(_F32).T, b_qkv[rows].astype(_F32)[None, :]

    wk0, bk0 = kv_w(0, 1)
    wk1, bk1 = kv_w(1, 1)
    wv0, bv0 = kv_w(0, 2)
    wv1, bv1 = kv_w(1, 2)

    wproj = W_proj.astype(_F32).T
    bproj = b_proj.astype(_F32)[None, :]

    bpe = b_pe.astype(_F32)[None, :]

    return pl.pallas_call(
        _kernel,
        out_shape=jax.ShapeDtypeStruct((_T, _DIM), _F32),
    )(x, posp, wpe, bpe, wq, bq,
      wk0, wk1, wv0, wv1, bk0, bk1, bv0, bv1, wproj, bproj)


# speedup vs baseline: 37109.4404x; 1.0161x over previous
import jax
import jax.numpy as jnp
import numpy as np
from jax.experimental import pallas as pl
from jax.experimental.pallas import tpu as pltpu

_DIM = 128
_H = 8
_E = _DIM // _H
_M = 16
_TOPK = 2
_T = 4096
_B = 2
_N = _T // (_B * _M)
_NB = _T // _M
_TB = _T // _B
_SCALE = 1.0 / (_E ** 0.5)

_F32 = jnp.float32


def _nt(a, b):
    return jax.lax.dot_general(
        a, b, (((1,), (1,)), ((), ())), preferred_element_type=_F32)


def _nn(a, b):
    return jax.lax.dot_general(
        a, b, (((1,), (0,)), ((), ())), preferred_element_type=_F32)


def _kernel(x_ref, posp_ref, wpe_ref, bpe_ref, wq_ref, bq_ref,
            wk0_ref, wk1_ref, wv0_ref, wv1_ref,
            bk0_ref, bk1_ref, bv0_ref, bv1_ref,
            wproj_ref, bproj_ref, out_ref):
    posp = posp_ref[:]
    pp = _nn(posp, wpe_ref[:])

    pp3 = pp.reshape(_NB, _M, _DIM)
    rel = (pp3 - jnp.mean(pp3, axis=1, keepdims=True)).reshape(_T, _DIM)
    xp = x_ref[:] + rel + bpe_ref[:]

    q = _nn(xp, wq_ref[:]) + bq_ref[:]

    wproj = wproj_ref[:]
    for bb in range(_B):
        base = bb * _TB
        xkv = xp[base:base + _H * _M, :]
        kmat0 = _nn(xkv, wk0_ref[:]) + bk0_ref[:]
        kmat1 = _nn(xkv, wk1_ref[:]) + bk1_ref[:]
        vmat0 = _nn(xkv, wv0_ref[:]) + bv0_ref[:]
        vmat1 = _nn(xkv, wv1_ref[:]) + bv1_ref[:]
        qb = q[base:base + _TB, :]
        acc = jnp.broadcast_to(bproj_ref[:], (_TB, _DIM))
        for h in range(_H):
            qh = qb[:, h * _E:(h + 1) * _E]
            k0 = kmat0[h * _M:(h + 1) * _M, :]
            k1 = kmat1[h * _M:(h + 1) * _M, :]
            v0 = vmat0[h * _M:(h + 1) * _M, :]
            v1 = vmat1[h * _M:(h + 1) * _M, :]
            s0 = _nt(qh, k0) * _SCALE
            s1 = _nt(qh, k1) * _SCALE
            mx = jnp.maximum(jnp.max(s0, axis=1, keepdims=True),
                             jnp.max(s1, axis=1, keepdims=True))
            e0 = jnp.exp(s0 - mx)
            e1 = jnp.exp(s1 - mx)
            den = (jnp.sum(e0, axis=1, keepdims=True) +
                   jnp.sum(e1, axis=1, keepdims=True))
            o = (_nn(e0, v0) + _nn(e1, v1)) / den
            acc = acc + _nn(o, wproj[h * _E:(h + 1) * _E, :])
        out_ref[base:base + _TB, :] = acc


def kernel(x, pos, W_pe, b_pe, W_qkv, b_qkv, W_proj, b_proj, num_batches):
    del num_batches
    x = x.astype(_F32)
    dimy = pos.shape[-1]

    posp = jnp.pad(pos.astype(_F32), ((0, 0), (0, _DIM - dimy)))
    wpe = jnp.pad(W_pe.astype(_F32).T, ((0, _DIM - dimy), (0, 0)))

    hh = np.arange(_H)[:, None]
    ee = np.arange(_E)[None, :]
    q_rows = (hh * (_E * 3) + ee * 3 + 0).reshape(-1)
    wq = W_qkv[q_rows].astype(_F32).T
    bq = b_qkv[q_rows].astype(_F32)[None, :]

    def kv_w(head, s):
        rows = (head * (_E * 3) + np.arange(_E) * 3 + s)
        return W_qkv[rows].astype(_F32).T, b_qkv[rows].astype(_F32)[None, :]

    wk0, bk0 = kv_w(0, 1)
    wk1, bk1 = kv_w(1, 1)
    wv0, bv0 = kv_w(0, 2)
    wv1, bv1 = kv_w(1, 2)

    wproj = W_proj.astype(_F32).T
    bproj = b_proj.astype(_F32)[None, :]

    bpe = b_pe.astype(_F32)[None, :]

    return pl.pallas_call(
        _kernel,
        out_shape=jax.ShapeDtypeStruct((_T, _DIM), _F32),
    )(x, posp, wpe, bpe, wq, bq,
      wk0, wk1, wv0, wv1, bk0, bk1, bv0, bv1, wproj, bproj)


# speedup vs baseline: 63173.8050x; 1.7024x over previous
import jax
import jax.numpy as jnp
import numpy as np
from jax.experimental import pallas as pl
from jax.experimental.pallas import tpu as pltpu

_DIM = 128
_H = 8
_E = _DIM // _H
_M = 16
_TOPK = 2
_T = 4096
_B = 2
_N = _T // (_B * _M)
_NB = _T // _M
_TB = _T // _B
_SCALE = 1.0 / (_E ** 0.5)

_F32 = jnp.float32


def _nt(a, b):
    return jax.lax.dot_general(
        a, b, (((1,), (1,)), ((), ())), preferred_element_type=_F32)


def _nn(a, b):
    return jax.lax.dot_general(
        a, b, (((1,), (0,)), ((), ())), preferred_element_type=_F32)


def _tn(a, b):
    return jax.lax.dot_general(
        a, b, (((0,), (0,)), ((), ())), preferred_element_type=_F32)


def _kernel(x_ref, posp_ref, wpe_ref, bpe_ref, wq_ref, bq_ref,
            wk0_ref, wk1_ref, wv0_ref, wv1_ref,
            bk0_ref, bk1_ref, bv0_ref, bv1_ref,
            wproj_ref, bproj_ref, out_ref):
    posp = posp_ref[:]
    pp = _nn(posp, wpe_ref[:])

    pp3 = pp.reshape(_NB, _M, _DIM)
    rel = (pp3 - jnp.mean(pp3, axis=1, keepdims=True)).reshape(_T, _DIM)
    xp = x_ref[:] + rel + bpe_ref[:]

    q = _nn(xp, wq_ref[:]) + bq_ref[:]

    wproj_raw = wproj_ref[:]
    for bb in range(_B):
        base = bb * _TB
        xkv = xp[base:base + _H * _M, :]
        kmat0 = _nn(xkv, wk0_ref[:]) + bk0_ref[:]
        kmat1 = _nn(xkv, wk1_ref[:]) + bk1_ref[:]
        vmat0 = _nn(xkv, wv0_ref[:]) + bv0_ref[:]
        vmat1 = _nn(xkv, wv1_ref[:]) + bv1_ref[:]
        qb = q[base:base + _TB, :]
        o_rows = []
        for h in range(_H):
            qh = qb[:, h * _E:(h + 1) * _E]
            kcat = jnp.concatenate(
                [kmat0[h * _M:(h + 1) * _M, :],
                 kmat1[h * _M:(h + 1) * _M, :]], axis=0)
            vcat = jnp.concatenate(
                [vmat0[h * _M:(h + 1) * _M, :],
                 vmat1[h * _M:(h + 1) * _M, :]], axis=0)
            st = _nt(kcat, qh) * _SCALE
            mx = jnp.max(st, axis=0, keepdims=True)
            e = jnp.exp(st - mx)
            den = jnp.sum(e, axis=0, keepdims=True)
            o_rows.append(_tn(vcat, e) / den)
        attn_t = jnp.concatenate(o_rows, axis=0)
        out_t = _nn(wproj_raw, attn_t)
        out_ref[base:base + _TB, :] = out_t.T + bproj_ref[:]


def kernel(x, pos, W_pe, b_pe, W_qkv, b_qkv, W_proj, b_proj, num_batches):
    del num_batches
    x = x.astype(_F32)
    dimy = pos.shape[-1]

    posp = jnp.pad(pos.astype(_F32), ((0, 0), (0, _DIM - dimy)))
    wpe = jnp.pad(W_pe.astype(_F32).T, ((0, _DIM - dimy), (0, 0)))

    hh = np.arange(_H)[:, None]
    ee = np.arange(_E)[None, :]
    q_rows = (hh * (_E * 3) + ee * 3 + 0).reshape(-1)
    wq = W_qkv[q_rows].astype(_F32).T
    bq = b_qkv[q_rows].astype(_F32)[None, :]

    def kv_w(head, s):
        rows = (head * (_E * 3) + np.arange(_E) * 3 + s)
        return W_qkv[rows].astype(_F32).T, b_qkv[rows].astype(_F32)[None, :]

    wk0, bk0 = kv_w(0, 1)
    wk1, bk1 = kv_w(1, 1)
    wv0, bv0 = kv_w(0, 2)
    wv1, bv1 = kv_w(1, 2)

    wproj = W_proj.astype(_F32)
    bproj = b_proj.astype(_F32)[None, :]

    bpe = b_pe.astype(_F32)[None, :]

    return pl.pallas_call(
        _kernel,
        out_shape=jax.ShapeDtypeStruct((_T, _DIM), _F32),
    )(x, posp, wpe, bpe, wq, bq,
      wk0, wk1, wv0, wv1, bk0, bk1, bv0, bv1, wproj, bproj)


# speedup vs baseline: 63503.1566x; 1.0052x over previous
import jax
import jax.numpy as jnp
import numpy as np
from jax.experimental import pallas as pl
from jax.experimental.pallas import tpu as pltpu

_DIM = 128
_H = 8
_E = _DIM // _H
_M = 16
_TOPK = 2
_T = 4096
_B = 2
_N = _T // (_B * _M)
_NB = _T // _M
_TB = _T // _B
_SCALE = 1.0 / (_E ** 0.5)

_F32 = jnp.float32


def _nt(a, b):
    return jax.lax.dot_general(
        a, b, (((1,), (1,)), ((), ())), preferred_element_type=_F32)


def _nn(a, b):
    return jax.lax.dot_general(
        a, b, (((1,), (0,)), ((), ())), preferred_element_type=_F32)


def _tn(a, b):
    return jax.lax.dot_general(
        a, b, (((0,), (0,)), ((), ())), preferred_element_type=_F32)


def _kernel(x_ref, posp_ref, wpe_ref, bpe_ref, wq_ref, bq_ref,
            wk0_ref, wk1_ref, wv0_ref, wv1_ref,
            bk0_ref, bk1_ref, bv0_ref, bv1_ref,
            wproj_ref, bproj_ref, out_ref):
    posp = posp_ref[:]
    pp = _nn(posp, wpe_ref[:])

    pp3 = pp.reshape(_NB, _M, _DIM)
    rel = (pp3 - jnp.mean(pp3, axis=1, keepdims=True)).reshape(_T, _DIM)
    xp = x_ref[:] + rel + bpe_ref[:]

    q = _nn(xp, wq_ref[:]) + bq_ref[:]

    wproj_t = wproj_ref[:]
    for bb in range(_B):
        base = bb * _TB
        xkv = xp[base:base + _H * _M, :]
        kmat0 = _nn(xkv, wk0_ref[:]) + bk0_ref[:]
        kmat1 = _nn(xkv, wk1_ref[:]) + bk1_ref[:]
        vmat0 = _nn(xkv, wv0_ref[:]) + bv0_ref[:]
        vmat1 = _nn(xkv, wv1_ref[:]) + bv1_ref[:]
        qb = q[base:base + _TB, :]
        o_rows = []
        for h in range(_H):
            qh = qb[:, h * _E:(h + 1) * _E]
            kcat = jnp.concatenate(
                [kmat0[h * _M:(h + 1) * _M, :],
                 kmat1[h * _M:(h + 1) * _M, :]], axis=0)
            vcat = jnp.concatenate(
                [vmat0[h * _M:(h + 1) * _M, :],
                 vmat1[h * _M:(h + 1) * _M, :]], axis=0)
            st = _nt(kcat, qh)
            mx = jnp.max(st, axis=0, keepdims=True)
            e = jnp.exp(st - mx)
            rden = 1.0 / jnp.sum(e, axis=0, keepdims=True)
            o_rows.append(_tn(vcat, e) * rden)
        attn_t = jnp.concatenate(o_rows, axis=0)
        out_ref[base:base + _TB, :] = _tn(attn_t, wproj_t) + bproj_ref[:]


def kernel(x, pos, W_pe, b_pe, W_qkv, b_qkv, W_proj, b_proj, num_batches):
    del num_batches
    x = x.astype(_F32)
    dimy = pos.shape[-1]

    posp = jnp.pad(pos.astype(_F32), ((0, 0), (0, 8 - dimy)))
    wpe = jnp.pad(W_pe.astype(_F32).T, ((0, 8 - dimy), (0, 0)))

    hh = np.arange(_H)[:, None]
    ee = np.arange(_E)[None, :]
    q_rows = (hh * (_E * 3) + ee * 3 + 0).reshape(-1)
    wq = W_qkv[q_rows].astype(_F32).T
    bq = b_qkv[q_rows].astype(_F32)[None, :]

    def kv_w(head, s, scale=1.0):
        rows = (head * (_E * 3) + np.arange(_E) * 3 + s)
        return (W_qkv[rows].astype(_F32).T * scale,
                b_qkv[rows].astype(_F32)[None, :] * scale)

    wk0, bk0 = kv_w(0, 1, _SCALE)
    wk1, bk1 = kv_w(1, 1, _SCALE)
    wv0, bv0 = kv_w(0, 2)
    wv1, bv1 = kv_w(1, 2)

    wproj = W_proj.astype(_F32).T
    bproj = b_proj.astype(_F32)[None, :]

    bpe = b_pe.astype(_F32)[None, :]

    return pl.pallas_call(
        _kernel,
        out_shape=jax.ShapeDtypeStruct((_T, _DIM), _F32),
    )(x, posp, wpe, bpe, wq, bq,
      wk0, wk1, wv0, wv1, bk0, bk1, bv0, bv1, wproj, bproj)


# speedup vs baseline: 108967.0286x; 1.7159x over previous
import jax
import jax.numpy as jnp
from jax.experimental import pallas as pl
from jax.experimental.pallas import tpu as pltpu

_DIM = 128
_H = 8
_E = _DIM // _H
_M = 16
_T = 4096
_B = 2
_NB = _T // _M
_TB = _T // _B
_QKV = 3 * _DIM
_SCALE = 1.0 / (_E ** 0.5)

_F32 = jnp.float32


def _nt(a, b):
    return jax.lax.dot_general(
        a, b, (((1,), (1,)), ((), ())), preferred_element_type=_F32)


def _nn(a, b):
    return jax.lax.dot_general(
        a, b, (((1,), (0,)), ((), ())), preferred_element_type=_F32)


def _tn(a, b):
    return jax.lax.dot_general(
        a, b, (((0,), (0,)), ((), ())), preferred_element_type=_F32)


def _sel(rows, target_fn, scale=1.0):
    ri = jax.lax.broadcasted_iota(jnp.int32, (rows, _QKV), 0)
    ci = jax.lax.broadcasted_iota(jnp.int32, (rows, _QKV), 1)
    return jnp.where(ci == target_fn(ri), scale, 0.0).astype(_F32)


def _kernel(x_ref, pos_ref, wpe_ref, bpe_ref, wqkv_ref, bqkv_ref,
            wproj_ref, bproj_ref, out_ref):
    pp = _nt(pos_ref[:], wpe_ref[:])

    pp3 = pp.reshape(_NB, _M, _DIM)
    rel = (pp3 - jnp.mean(pp3, axis=1, keepdims=True)).reshape(_T, _DIM)
    xp = x_ref[:] + rel + bpe_ref[:]

    wqkv = wqkv_ref[:]
    bqkv = bqkv_ref[:]
    sel_q = _sel(_DIM, lambda i: 3 * i)
    w2q = _nn(sel_q, wqkv)
    bq = _nt(bqkv, sel_q)
    q = _nt(xp, w2q) + bq

    sel_k0 = _sel(_E, lambda i: 3 * i + 1, _SCALE)
    sel_k1 = _sel(_E, lambda i: 48 + 3 * i + 1, _SCALE)
    sel_v0 = _sel(_E, lambda i: 3 * i + 2)
    sel_v1 = _sel(_E, lambda i: 48 + 3 * i + 2)
    w2k0, bk0 = _nn(sel_k0, wqkv), _nt(bqkv, sel_k0)
    w2k1, bk1 = _nn(sel_k1, wqkv), _nt(bqkv, sel_k1)
    w2v0, bv0 = _nn(sel_v0, wqkv), _nt(bqkv, sel_v0)
    w2v1, bv1 = _nn(sel_v1, wqkv), _nt(bqkv, sel_v1)

    wproj_t = wproj_ref[:].T
    for bb in range(_B):
        base = bb * _TB
        xkv = xp[base:base + _H * _M, :]
        kmat0 = _nt(xkv, w2k0) + bk0
        kmat1 = _nt(xkv, w2k1) + bk1
        vmat0 = _nt(xkv, w2v0) + bv0
        vmat1 = _nt(xkv, w2v1) + bv1
        qb = q[base:base + _TB, :]
        o_rows = []
        for h in range(_H):
            qh = qb[:, h * _E:(h + 1) * _E]
            kcat = jnp.concatenate(
                [kmat0[h * _M:(h + 1) * _M, :],
                 kmat1[h * _M:(h + 1) * _M, :]], axis=0)
            vcat = jnp.concatenate(
                [vmat0[h * _M:(h + 1) * _M, :],
                 vmat1[h * _M:(h + 1) * _M, :]], axis=0)
            st = _nt(kcat, qh)
            mx = jnp.max(st, axis=0, keepdims=True)
            e = jnp.exp(st - mx)
            rden = 1.0 / jnp.sum(e, axis=0, keepdims=True)
            o_rows.append(_tn(vcat, e) * rden)
        attn_t = jnp.concatenate(o_rows, axis=0)
        out_ref[base:base + _TB, :] = _tn(attn_t, wproj_t) + bproj_ref[:]


def kernel(x, pos, W_pe, b_pe, W_qkv, b_qkv, W_proj, b_proj, num_batches):
    del num_batches
    return pl.pallas_call(
        _kernel,
        out_shape=jax.ShapeDtypeStruct((_T, _DIM), _F32),
    )(x.astype(_F32), pos.astype(_F32), W_pe.astype(_F32),
      b_pe.astype(_F32)[None, :], W_qkv.astype(_F32),
      b_qkv.astype(_F32)[None, :], W_proj.astype(_F32),
      b_proj.astype(_F32)[None, :])


# speedup vs baseline: 109326.7544x; 1.0033x over previous
import jax
import jax.numpy as jnp
from jax.experimental import pallas as pl
from jax.experimental.pallas import tpu as pltpu

_DIM = 128
_H = 8
_E = _DIM // _H
_M = 16
_T = 4096
_B = 2
_NB = _T // _M
_TB = _T // _B
_QKV = 3 * _DIM
_SCALE = 1.0 / (_E ** 0.5)

_F32 = jnp.float32


def _nt(a, b):
    return jax.lax.dot_general(
        a, b, (((1,), (1,)), ((), ())), preferred_element_type=_F32)


def _nn(a, b):
    return jax.lax.dot_general(
        a, b, (((1,), (0,)), ((), ())), preferred_element_type=_F32)


def _tn(a, b):
    return jax.lax.dot_general(
        a, b, (((0,), (0,)), ((), ())), preferred_element_type=_F32)


def _sel(rows, target_fn, scale=1.0):
    ri = jax.lax.broadcasted_iota(jnp.int32, (rows, _QKV), 0)
    ci = jax.lax.broadcasted_iota(jnp.int32, (rows, _QKV), 1)
    return jnp.where(ci == target_fn(ri), scale, 0.0).astype(_F32)


def _kernel(x_ref, pos_ref, wpe_ref, bpe_ref, wqkv_ref, bqkv_ref,
            wproj_ref, bproj_ref, out_ref):
    pp = _nt(pos_ref[:], wpe_ref[:])

    pp3 = pp.reshape(_TB // _M, _M, _DIM)
    rel = (pp3 - jnp.mean(pp3, axis=1, keepdims=True)).reshape(_TB, _DIM)
    xp = x_ref[:] + rel + bpe_ref[:]

    wqkv = wqkv_ref[:]
    bqkv = bqkv_ref[:]
    sel_q = _sel(_DIM, lambda i: 3 * i)
    w2q = _nn(sel_q, wqkv)
    bq = _nt(bqkv, sel_q)
    q = _nt(xp, w2q) + bq

    sel_k0 = _sel(_E, lambda i: 3 * i + 1, _SCALE)
    sel_k1 = _sel(_E, lambda i: 48 + 3 * i + 1, _SCALE)
    sel_v0 = _sel(_E, lambda i: 3 * i + 2)
    sel_v1 = _sel(_E, lambda i: 48 + 3 * i + 2)
    w2k0, bk0 = _nn(sel_k0, wqkv), _nt(bqkv, sel_k0)
    w2k1, bk1 = _nn(sel_k1, wqkv), _nt(bqkv, sel_k1)
    w2v0, bv0 = _nn(sel_v0, wqkv), _nt(bqkv, sel_v0)
    w2v1, bv1 = _nn(sel_v1, wqkv), _nt(bqkv, sel_v1)

    wproj_t = wproj_ref[:].T
    xkv = xp[0:_H * _M, :]
    kmat0 = _nt(xkv, w2k0) + bk0
    kmat1 = _nt(xkv, w2k1) + bk1
    vmat0 = _nt(xkv, w2v0) + bv0
    vmat1 = _nt(xkv, w2v1) + bv1
    o_rows = []
    for h in range(_H):
        qh = q[:, h * _E:(h + 1) * _E]
        kcat = jnp.concatenate(
            [kmat0[h * _M:(h + 1) * _M, :],
             kmat1[h * _M:(h + 1) * _M, :]], axis=0)
        vcat = jnp.concatenate(
            [vmat0[h * _M:(h + 1) * _M, :],
             vmat1[h * _M:(h + 1) * _M, :]], axis=0)
        st = _nt(kcat, qh)
        mx = jnp.max(st, axis=0, keepdims=True)
        e = jnp.exp(st - mx)
        rden = 1.0 / jnp.sum(e, axis=0, keepdims=True)
        o_rows.append(_tn(vcat, e) * rden)
    attn_t = jnp.concatenate(o_rows, axis=0)
    out_ref[:] = _tn(attn_t, wproj_t) + bproj_ref[:]


def kernel(x, pos, W_pe, b_pe, W_qkv, b_qkv, W_proj, b_proj, num_batches):
    del num_batches
    full = lambda shape: pl.BlockSpec(shape, lambda i: (0, 0))
    return pl.pallas_call(
        _kernel,
        grid=(_B,),
        in_specs=[
            pl.BlockSpec((_TB, _DIM), lambda i: (i, 0)),
            pl.BlockSpec((_TB, 3), lambda i: (i, 0)),
            full((_DIM, 3)),
            full((1, _DIM)),
            full((_QKV, _DIM)),
            full((1, _QKV)),
            full((_DIM, _DIM)),
            full((1, _DIM)),
        ],
        out_specs=pl.BlockSpec((_TB, _DIM), lambda i: (i, 0)),
        out_shape=jax.ShapeDtypeStruct((_T, _DIM), _F32),
        compiler_params=pltpu.CompilerParams(
            dimension_semantics=("parallel",)),
    )(x.astype(_F32), pos.astype(_F32), W_pe.astype(_F32),
      b_pe.astype(_F32)[None, :], W_qkv.astype(_F32),
      b_qkv.astype(_F32)[None, :], W_proj.astype(_F32),
      b_proj.astype(_F32)[None, :])


# speedup vs baseline: 113278.1377x; 1.0361x over previous
import jax
import jax.numpy as jnp
from jax.experimental import pallas as pl
from jax.experimental.pallas import tpu as pltpu

_DIM = 128
_H = 8
_E = _DIM // _H
_M = 16
_T = 4096
_B = 2
_NB = _T // _M
_TB = _T // _B
_QKV = 3 * _DIM
_SCALE = 1.0 / (_E ** 0.5)

_F32 = jnp.float32
_BF16 = jnp.bfloat16


def _bf(a):
    return a.astype(_BF16)


def _nt(a, b):
    return jax.lax.dot_general(
        a, b, (((1,), (1,)), ((), ())), preferred_element_type=_F32)


def _nn(a, b):
    return jax.lax.dot_general(
        a, b, (((1,), (0,)), ((), ())), preferred_element_type=_F32)


def _tn(a, b):
    return jax.lax.dot_general(
        a, b, (((0,), (0,)), ((), ())), preferred_element_type=_F32)


def _sel(rows, target_fn, scale=1.0):
    ri = jax.lax.broadcasted_iota(jnp.int32, (rows, _QKV), 0)
    ci = jax.lax.broadcasted_iota(jnp.int32, (rows, _QKV), 1)
    return jnp.where(ci == target_fn(ri), scale, 0.0).astype(_F32)


def _kernel(x_ref, pos_ref, wpe_ref, bpe_ref, wqkv_ref, bqkv_ref,
            wproj_ref, bproj_ref, out_ref):
    pp = _nt(pos_ref[:], wpe_ref[:])

    pp3 = pp.reshape(_TB // _M, _M, _DIM)
    rel = (pp3 - jnp.mean(pp3, axis=1, keepdims=True)).reshape(_TB, _DIM)
    xp = x_ref[:] + rel + bpe_ref[:]

    wqkv = wqkv_ref[:]
    bqkv = bqkv_ref[:]
    sel_q = _sel(_DIM, lambda i: 3 * i)
    w2q = _bf(_nn(sel_q, wqkv))
    bq = _nt(bqkv, sel_q)
    xp_bf = _bf(xp)
    q = _nt(xp_bf, w2q) + bq

    sel_k0 = _sel(_E, lambda i: 3 * i + 1, _SCALE)
    sel_k1 = _sel(_E, lambda i: 48 + 3 * i + 1, _SCALE)
    sel_v0 = _sel(_E, lambda i: 3 * i + 2)
    sel_v1 = _sel(_E, lambda i: 48 + 3 * i + 2)
    w2k0, bk0 = _bf(_nn(sel_k0, wqkv)), _nt(bqkv, sel_k0)
    w2k1, bk1 = _bf(_nn(sel_k1, wqkv)), _nt(bqkv, sel_k1)
    w2v0, bv0 = _bf(_nn(sel_v0, wqkv)), _nt(bqkv, sel_v0)
    w2v1, bv1 = _bf(_nn(sel_v1, wqkv)), _nt(bqkv, sel_v1)

    wproj_t = _bf(wproj_ref[:].T)
    xkv = xp_bf[0:_H * _M, :]
    kmat0 = _bf(_nt(xkv, w2k0) + bk0)
    kmat1 = _bf(_nt(xkv, w2k1) + bk1)
    vmat0 = _bf(_nt(xkv, w2v0) + bv0)
    vmat1 = _bf(_nt(xkv, w2v1) + bv1)
    qbf = _bf(q)
    o_rows = []
    for h in range(_H):
        qh = qbf[:, h * _E:(h + 1) * _E]
        kcat = jnp.concatenate(
            [kmat0[h * _M:(h + 1) * _M, :],
             kmat1[h * _M:(h + 1) * _M, :]], axis=0)
        vcat = jnp.concatenate(
            [vmat0[h * _M:(h + 1) * _M, :],
             vmat1[h * _M:(h + 1) * _M, :]], axis=0)
        st = _nt(kcat, qh)
        mx = jnp.max(st, axis=0, keepdims=True)
        e = jnp.exp(st - mx)
        rden = 1.0 / jnp.sum(e, axis=0, keepdims=True)
        o_rows.append(_tn(vcat, _bf(e)) * rden)
    attn_t = _bf(jnp.concatenate(o_rows, axis=0))
    out_ref[:] = _tn(attn_t, wproj_t) + bproj_ref[:]


def kernel(x, pos, W_pe, b_pe, W_qkv, b_qkv, W_proj, b_proj, num_batches):
    del num_batches
    full = lambda shape: pl.BlockSpec(shape, lambda i: (0, 0))
    return pl.pallas_call(
        _kernel,
        grid=(_B,),
        in_specs=[
            pl.BlockSpec((_TB, _DIM), lambda i: (i, 0)),
            pl.BlockSpec((_TB, 3), lambda i: (i, 0)),
            full((_DIM, 3)),
            full((1, _DIM)),
            full((_QKV, _DIM)),
            full((1, _QKV)),
            full((_DIM, _DIM)),
            full((1, _DIM)),
        ],
        out_specs=pl.BlockSpec((_TB, _DIM), lambda i: (i, 0)),
        out_shape=jax.ShapeDtypeStruct((_T, _DIM), _F32),
        compiler_params=pltpu.CompilerParams(
            dimension_semantics=("parallel",)),
    )(x.astype(_F32), pos.astype(_F32), W_pe.astype(_F32),
      b_pe.astype(_F32)[None, :], W_qkv.astype(_F32),
      b_qkv.astype(_F32)[None, :], W_proj.astype(_F32),
      b_proj.astype(_F32)[None, :])
